# baseline (speedup 1.0000x reference)
import functools

import jax
import jax.numpy as jnp
from jax import lax
from jax.experimental import pallas as pl
from jax.experimental.pallas import tpu as pltpu
from jax.experimental.pallas import tpu_sc as plsc

_LANES = 16
_ROW_PAD = 64
_NUM_WORKERS = 32
_CHUNK = 512
_OUT_STRIDE = _CHUNK + 16


def _cp_feature_call(xyz_t, tbl_flat, n, ncomp, grid):
    rows = grid + 1
    ax_stride = rows * _ROW_PAD
    ppt = n // _NUM_WORKERS
    nchunks = ppt // _CHUNK
    scale = 0.5 * (grid - 1)
    nblk = ncomp // _LANES

    mesh = plsc.VectorSubcoreMesh(core_axis_name="c", subcore_axis_name="s")

    @functools.partial(
        pl.kernel,
        mesh=mesh,
        compiler_params=pltpu.CompilerParams(needs_layout_passes=False),
        out_type=jax.ShapeDtypeStruct((ncomp, n), jnp.float32),
        scratch_types=[
            pltpu.VMEM((3 * ax_stride,), jnp.float32),
            pltpu.VMEM((3, _CHUNK), jnp.float32),
            pltpu.VMEM((3, _CHUNK), jnp.float32),
            pltpu.VMEM((ncomp, _OUT_STRIDE), jnp.float32),
            pltpu.VMEM((ncomp, _OUT_STRIDE), jnp.float32),
            pltpu.VMEM((_LANES, 48), jnp.float32),
            pltpu.SemaphoreType.DMA,
            pltpu.SemaphoreType.DMA,
            pltpu.SemaphoreType.DMA,
            pltpu.SemaphoreType.DMA,
        ],
    )
    def cp_kernel(xyz_hbm, tbl_hbm, out_hbm, tbl_v, xyz_v0, xyz_v1,
                  out_v0, out_v1, pad_v, sin0, sin1, sout0, sout1):
        wid = lax.axis_index("s") * 2 + lax.axis_index("c")
        base = wid * ppt
        pltpu.sync_copy(tbl_hbm, tbl_v)

        xbufs, obufs = [xyz_v0, xyz_v1], [out_v0, out_v1]
        sins, souts = [sin0, sin1], [sout0, sout1]

        def in_copy(j, buf, sem):
            cb = base + j * _CHUNK
            return pltpu.make_async_copy(
                xyz_hbm.at[:, pl.ds(cb, _CHUNK)], buf, sem
            )

        def out_copy(j, buf, sem):
            cb = base + j * _CHUNK
            return pltpu.make_async_copy(
                buf.at[:, pl.ds(0, _CHUNK)],
                out_hbm.at[:, pl.ds(cb, _CHUNK)],
                sem,
            )

        gl_iota = lax.iota(jnp.int32, _LANES)
        msk_c = [((gl_iota >> st) & 1) == 1 for st in range(4)]
        idx_m = [(gl_iota - (1 << st)) % _LANES for st in range(4)]
        idx_p = [(gl_iota + (1 << st)) % _LANES for st in range(4)]

        def chunk_work(j, xyz_v, out_v):

            def group_body(gi, c2):
                s = gi * _LANES
                offv, wv = [], []
                for a in range(3):
                    v = xyz_v[a, pl.ds(s, _LANES)]
                    pos = (v + 1.0) * scale
                    i0 = pos.astype(jnp.int32)
                    i0 = jnp.minimum(jnp.maximum(i0, 0), grid - 1)
                    wv.append(pos - i0.astype(jnp.float32))
                    offv.append(i0 * _ROW_PAD + a * ax_stride)

                lane_iota = lax.iota(jnp.int32, _LANES)

                def point_vals(p):
                    offs = [offv[a][p] for a in range(3)]
                    wts = [wv[a][p] for a in range(3)]
                    res = []
                    for k in range(nblk):
                        r = None
                        for a in range(3):
                            o = offs[a] + k * _LANES
                            lo = tbl_v[pl.ds(o, _LANES)]
                            hi = tbl_v[pl.ds(o + _ROW_PAD, _LANES)]
                            t = lo + wts[a] * (hi - lo)
                            r = t if r is None else r * t
                        res.append(r)
                    return res

                allres = [point_vals(p) for p in range(_LANES)]

                for k in range(nblk):
                    v = [allres[p][k] for p in range(_LANES)]
                    for st in range(4):
                        bit = 1 << st
                        mask = msk_c[st]
                        nv = [None] * _LANES
                        for p0 in range(_LANES):
                            if p0 & bit:
                                continue
                            p1 = p0 | bit
                            va, vb = v[p0], v[p1]
                            bp = vb.at[idx_m[st]].get(mode="promise_in_bounds")
                            ap = va.at[idx_p[st]].get(mode="promise_in_bounds")
                            nv[p0] = jnp.where(mask, bp, va)
                            nv[p1] = jnp.where(mask, vb, ap)
                        v = nv
                    for c in range(_LANES):
                        out_v[k * _LANES + c, pl.ds(s, _LANES)] = v[c]
                return c2

            lax.fori_loop(0, _CHUNK // _LANES, group_body, 0)

        in_copy(0, xbufs[0], sins[0]).start()

        def super_body(jj, carry):
            for ph in range(2):
                j = jj * 2 + ph
                nxt = (ph + 1) % 2

                @pl.when(j + 1 < nchunks)
                def _prefetch():
                    in_copy(j + 1, xbufs[nxt], sins[nxt]).start()

                in_copy(j, xbufs[ph], sins[ph]).wait()

                @pl.when(j >= 2)
                def _drain():
                    out_copy(j - 2, obufs[ph], souts[ph]).wait()

                chunk_work(j, xbufs[ph], obufs[ph])
                out_copy(j, obufs[ph], souts[ph]).start()
            return carry

        lax.fori_loop(0, nchunks // 2, super_body, 0)
        out_copy(nchunks - 2, obufs[0], souts[0]).wait()
        out_copy(nchunks - 1, obufs[1], souts[1]).wait()

    return cp_kernel(xyz_t, tbl_flat)


def kernel(xyz_sampled, param0, param1, param2):
    n = xyz_sampled.shape[0]
    ncomp = param0.shape[1]
    grid = param0.shape[2]

    xyz_t = xyz_sampled.T

    def prep(p):
        t = p[0, :, :, 0].T
        t = jnp.concatenate([t, t[-1:]], axis=0)
        t = jnp.pad(t, ((0, 0), (0, _ROW_PAD - ncomp)))
        return t

    tbl = jnp.stack([prep(param0), prep(param1), prep(param2)])
    return _cp_feature_call(xyz_t, tbl.reshape(-1), n, ncomp, grid)


# speedup vs baseline: 1.0629x; 1.0629x over previous
import functools

import jax
import jax.numpy as jnp
from jax import lax
from jax.experimental import pallas as pl
from jax.experimental.pallas import tpu as pltpu
from jax.experimental.pallas import tpu_sc as plsc

_LANES = 16
_ROW_PAD = 64
_NUM_WORKERS = 32
_CHUNK = 512
_OUT_STRIDE = _CHUNK + 16


def _cp_feature_call(xyz_t, tbl_flat, n, ncomp, grid):
    rows = grid + 1
    ax_stride = rows * _ROW_PAD
    ppt = n // _NUM_WORKERS
    nchunks = ppt // _CHUNK
    scale = 0.5 * (grid - 1)
    nblk = ncomp // _LANES

    mesh = plsc.VectorSubcoreMesh(core_axis_name="c", subcore_axis_name="s")

    @functools.partial(
        pl.kernel,
        mesh=mesh,
        compiler_params=pltpu.CompilerParams(needs_layout_passes=False),
        out_type=jax.ShapeDtypeStruct((ncomp, n), jnp.float32),
        scratch_types=[
            pltpu.VMEM((3 * ax_stride,), jnp.float32),
            pltpu.VMEM((3, _CHUNK), jnp.float32),
            pltpu.VMEM((3, _CHUNK), jnp.float32),
            pltpu.VMEM((ncomp, _OUT_STRIDE), jnp.float32),
            pltpu.VMEM((ncomp, _OUT_STRIDE), jnp.float32),
            pltpu.VMEM((_LANES, 48), jnp.float32),
            pltpu.SemaphoreType.DMA,
            pltpu.SemaphoreType.DMA,
            pltpu.SemaphoreType.DMA,
            pltpu.SemaphoreType.DMA,
        ],
    )
    def cp_kernel(xyz_hbm, tbl_hbm, out_hbm, tbl_v, xyz_v0, xyz_v1,
                  out_v0, out_v1, pad_v, sin0, sin1, sout0, sout1):
        wid = lax.axis_index("s") * 2 + lax.axis_index("c")
        base = wid * ppt
        pltpu.sync_copy(tbl_hbm, tbl_v)

        xbufs, obufs = [xyz_v0, xyz_v1], [out_v0, out_v1]
        sins, souts = [sin0, sin1], [sout0, sout1]

        def in_copy(j, buf, sem):
            cb = base + j * _CHUNK
            return pltpu.make_async_copy(
                xyz_hbm.at[:, pl.ds(cb, _CHUNK)], buf, sem
            )

        def out_copy(j, buf, sem):
            cb = base + j * _CHUNK
            return pltpu.make_async_copy(
                buf.at[:, pl.ds(0, _CHUNK)],
                out_hbm.at[:, pl.ds(cb, _CHUNK)],
                sem,
            )

        gl_iota = lax.iota(jnp.int32, _LANES)
        msk_c = [((gl_iota >> st) & 1) == 1 for st in range(4)]
        idx_m = [(gl_iota - (1 << st)) % _LANES for st in range(4)]
        idx_p = [(gl_iota + (1 << st)) % _LANES for st in range(4)]

        def chunk_work(j, xyz_v, out_v):

            @plsc.parallel_loop(0, _CHUNK // _LANES, unroll=1)
            def group_body(gi):
                s = gi * _LANES
                offv, wv = [], []
                for a in range(3):
                    v = xyz_v[a, pl.ds(s, _LANES)]
                    pos = (v + 1.0) * scale
                    i0 = pos.astype(jnp.int32)
                    i0 = jnp.minimum(jnp.maximum(i0, 0), grid - 1)
                    wv.append(pos - i0.astype(jnp.float32))
                    offv.append(i0 * _ROW_PAD + a * ax_stride)

                lane_iota = lax.iota(jnp.int32, _LANES)

                def point_vals(p):
                    idxs = [offv[a][p] + lane_iota for a in range(3)]
                    wts = [wv[a][p] for a in range(3)]
                    res = []
                    for k in range(nblk):
                        r = None
                        for a in range(3):
                            o = idxs[a] + k * _LANES
                            lo = plsc.load_gather(tbl_v, [o])
                            hi = plsc.load_gather(tbl_v, [o + _ROW_PAD])
                            t = lo + wts[a] * (hi - lo)
                            r = t if r is None else r * t
                        res.append(r)
                    return res

                allres = [point_vals(p) for p in range(_LANES)]

                for k in range(nblk):
                    v = [allres[p][k] for p in range(_LANES)]
                    for st in range(4):
                        bit = 1 << st
                        mask = msk_c[st]
                        nv = [None] * _LANES
                        for p0 in range(_LANES):
                            if p0 & bit:
                                continue
                            p1 = p0 | bit
                            va, vb = v[p0], v[p1]
                            bp = vb.at[idx_m[st]].get(mode="promise_in_bounds")
                            ap = va.at[idx_p[st]].get(mode="promise_in_bounds")
                            nv[p0] = jnp.where(mask, bp, va)
                            nv[p1] = jnp.where(mask, vb, ap)
                        v = nv
                    for c in range(_LANES):
                        out_v[k * _LANES + c, pl.ds(s, _LANES)] = v[c]

        in_copy(0, xbufs[0], sins[0]).start()

        def super_body(jj, carry):
            for ph in range(2):
                j = jj * 2 + ph
                nxt = (ph + 1) % 2

                @pl.when(j + 1 < nchunks)
                def _prefetch():
                    in_copy(j + 1, xbufs[nxt], sins[nxt]).start()

                in_copy(j, xbufs[ph], sins[ph]).wait()

                @pl.when(j >= 2)
                def _drain():
                    out_copy(j - 2, obufs[ph], souts[ph]).wait()

                chunk_work(j, xbufs[ph], obufs[ph])
                out_copy(j, obufs[ph], souts[ph]).start()
            return carry

        lax.fori_loop(0, nchunks // 2, super_body, 0)
        out_copy(nchunks - 2, obufs[0], souts[0]).wait()
        out_copy(nchunks - 1, obufs[1], souts[1]).wait()

    return cp_kernel(xyz_t, tbl_flat)


def kernel(xyz_sampled, param0, param1, param2):
    n = xyz_sampled.shape[0]
    ncomp = param0.shape[1]
    grid = param0.shape[2]

    xyz_t = xyz_sampled.T

    def prep(p):
        t = p[0, :, :, 0].T
        t = jnp.concatenate([t, t[-1:]], axis=0)
        t = jnp.pad(t, ((0, 0), (0, _ROW_PAD - ncomp)))
        return t

    tbl = jnp.stack([prep(param0), prep(param1), prep(param2)])
    return _cp_feature_call(xyz_t, tbl.reshape(-1), n, ncomp, grid)
